# baseline (speedup 1.0000x reference)
import functools

import jax
import jax.numpy as jnp
from jax.experimental import pallas as pl
from jax.experimental.pallas import tpu as pltpu


def _dist_kernel(f_ref, c_ref, o_ref, cs_ref, c2_ref):
    @pl.when(pl.program_id(0) == 0)
    def _():
        c = c_ref[...]
        cs_ref[...] = (-2.0 * c).astype(jnp.bfloat16)
        c2_ref[0:1, :] = jnp.sum(c * c, axis=1)[None, :]

    f = f_ref[...]
    f2 = jnp.sum(f * f, axis=1, keepdims=True)
    fc = jax.lax.dot_general(
        f.astype(jnp.bfloat16), cs_ref[...],
        (((1,), (1,)), ((), ())),
        preferred_element_type=jnp.float32,
    )
    o_ref[...] = (fc + f2) + c2_ref[0:1, :]


@functools.partial(jax.jit, static_argnames=("bm",))
def _dists(f, c, bm):
    M, D = f.shape
    K = c.shape[0]
    grid = (M // bm,)
    return pl.pallas_call(
        _dist_kernel,
        grid=grid,
        in_specs=[
            pl.BlockSpec((bm, D), lambda i: (i, 0)),
            pl.BlockSpec((K, D), lambda i: (0, 0)),
        ],
        out_specs=pl.BlockSpec((bm, K), lambda i: (i, 0)),
        out_shape=jax.ShapeDtypeStruct((M, K), jnp.float32),
        scratch_shapes=[
            pltpu.VMEM((K, D), jnp.bfloat16),
            pltpu.VMEM((8, K), jnp.float32),
        ],
        compiler_params=pltpu.CompilerParams(
            dimension_semantics=("arbitrary",),
        ),
    )(f, c)


def kernel(features, Ck):
    B, S, D = features.shape
    K = Ck.shape[1]
    f = features.reshape(B * S, D)
    c = Ck.reshape(K, D)
    dists = _dists(f, c, bm=3072)
    return dists.reshape(B, S, K)


# speedup vs baseline: 1.0611x; 1.0611x over previous
import functools

import jax
import jax.numpy as jnp
from jax.experimental import pallas as pl
from jax.experimental.pallas import tpu as pltpu

_CHUNKS = ((0, 512), (512, 1024), (1536, 2048), (3584, 2560), (6144, 3072))
_MAXROWS = 3072


def _dist_kernel(f_hbm, c_ref, o_hbm, fbuf0, fbuf1, obuf0, obuf1, cs_buf,
                 ld_sem, st_sem):
    c = c_ref[...]
    cs_buf[...] = (-2.0 * c).astype(jnp.bfloat16)
    c2 = jnp.sum(c * c, axis=1)[None, :]

    fbufs = (fbuf0, fbuf1)
    obufs = (obuf0, obuf1)

    loads = [
        pltpu.make_async_copy(
            f_hbm.at[pl.ds(r0, sz), :],
            fbufs[i % 2].at[pl.ds(0, sz), :],
            ld_sem.at[i % 2],
        )
        for i, (r0, sz) in enumerate(_CHUNKS)
    ]
    stores = []
    loads[0].start()
    for i, (r0, sz) in enumerate(_CHUNKS):
        if i + 1 < len(_CHUNKS):
            loads[i + 1].start()
        loads[i].wait()
        if i >= 2:
            stores[i - 2].wait()
        f = fbufs[i % 2][pl.ds(0, sz), :]
        f2 = jnp.sum(f * f, axis=1, keepdims=True)
        fc = jax.lax.dot_general(
            f.astype(jnp.bfloat16), cs_buf[...],
            (((1,), (1,)), ((), ())),
            preferred_element_type=jnp.float32,
        )
        obufs[i % 2][pl.ds(0, sz), :] = (fc + f2) + c2
        st = pltpu.make_async_copy(
            obufs[i % 2].at[pl.ds(0, sz), :],
            o_hbm.at[pl.ds(r0, sz), :],
            st_sem.at[i % 2],
        )
        st.start()
        stores.append(st)
    stores[-2].wait()
    stores[-1].wait()


@jax.jit
def _dists(f, c):
    M, D = f.shape
    K = c.shape[0]
    return pl.pallas_call(
        _dist_kernel,
        in_specs=[
            pl.BlockSpec(memory_space=pl.ANY),
            pl.BlockSpec((K, D), lambda: (0, 0)),
        ],
        out_specs=pl.BlockSpec(memory_space=pl.ANY),
        out_shape=jax.ShapeDtypeStruct((M, K), jnp.float32),
        scratch_shapes=[
            pltpu.VMEM((_MAXROWS, D), jnp.float32),
            pltpu.VMEM((_MAXROWS, D), jnp.float32),
            pltpu.VMEM((_MAXROWS, K), jnp.float32),
            pltpu.VMEM((_MAXROWS, K), jnp.float32),
            pltpu.VMEM((K, D), jnp.bfloat16),
            pltpu.SemaphoreType.DMA((2,)),
            pltpu.SemaphoreType.DMA((2,)),
        ],
    )(f, c)


def kernel(features, Ck):
    B, S, D = features.shape
    K = Ck.shape[1]
    f = features.reshape(B * S, D)
    c = Ck.reshape(K, D)
    dists = _dists(f, c)
    return dists.reshape(B, S, K)


# speedup vs baseline: 1.0990x; 1.0358x over previous
import functools

import jax
import jax.numpy as jnp
from jax.experimental import pallas as pl
from jax.experimental.pallas import tpu as pltpu

_CHUNKS = ((0, 256), (256, 512), (768, 1024), (1792, 2048), (3840, 2560),
           (6400, 2816))
_MAXROWS = 2816


def _dist_kernel(f_hbm, c_ref, o_hbm, fbuf0, fbuf1, fbuf2, obuf0, obuf1,
                 obuf2, cs_buf, ld_sem, st_sem):
    fbufs = (fbuf0, fbuf1, fbuf2)
    obufs = (obuf0, obuf1, obuf2)

    loads = [
        pltpu.make_async_copy(
            f_hbm.at[pl.ds(r0, sz), :],
            fbufs[i % 3].at[pl.ds(0, sz), :],
            ld_sem.at[i % 3],
        )
        for i, (r0, sz) in enumerate(_CHUNKS)
    ]
    loads[0].start()
    loads[1].start()

    c = c_ref[...]
    cs_buf[...] = (-2.0 * c).astype(jnp.bfloat16)
    c2 = jnp.sum(c * c, axis=1)[None, :]

    stores = []
    for i, (r0, sz) in enumerate(_CHUNKS):
        if i + 2 < len(_CHUNKS):
            loads[i + 2].start()
        loads[i].wait()
        if i >= 3:
            stores[i - 3].wait()
        f = fbufs[i % 3][pl.ds(0, sz), :]
        f2 = jnp.sum(f * f, axis=1, keepdims=True)
        fc = jax.lax.dot_general(
            f.astype(jnp.bfloat16), cs_buf[...],
            (((1,), (1,)), ((), ())),
            preferred_element_type=jnp.float32,
        )
        obufs[i % 3][pl.ds(0, sz), :] = (fc + f2) + c2
        st = pltpu.make_async_copy(
            obufs[i % 3].at[pl.ds(0, sz), :],
            o_hbm.at[pl.ds(r0, sz), :],
            st_sem.at[i % 3],
        )
        st.start()
        stores.append(st)
    stores[-3].wait()
    stores[-2].wait()
    stores[-1].wait()


@jax.jit
def _dists(f, c):
    M, D = f.shape
    K = c.shape[0]
    return pl.pallas_call(
        _dist_kernel,
        in_specs=[
            pl.BlockSpec(memory_space=pl.ANY),
            pl.BlockSpec((K, D), lambda: (0, 0)),
        ],
        out_specs=pl.BlockSpec(memory_space=pl.ANY),
        out_shape=jax.ShapeDtypeStruct((M, K), jnp.float32),
        scratch_shapes=[
            pltpu.VMEM((_MAXROWS, D), jnp.float32),
            pltpu.VMEM((_MAXROWS, D), jnp.float32),
            pltpu.VMEM((_MAXROWS, D), jnp.float32),
            pltpu.VMEM((_MAXROWS, K), jnp.float32),
            pltpu.VMEM((_MAXROWS, K), jnp.float32),
            pltpu.VMEM((_MAXROWS, K), jnp.float32),
            pltpu.VMEM((K, D), jnp.bfloat16),
            pltpu.SemaphoreType.DMA((3,)),
            pltpu.SemaphoreType.DMA((3,)),
        ],
    )(f, c)


def kernel(features, Ck):
    B, S, D = features.shape
    K = Ck.shape[1]
    f = features.reshape(B * S, D)
    c = Ck.reshape(K, D)
    dists = _dists(f, c)
    return dists.reshape(B, S, K)
